# baseline (speedup 1.0000x reference)
import jax
import jax.numpy as jnp
import numpy as np
from jax.experimental import pallas as pl
from jax.experimental.pallas import tpu as pltpu

_B, _L, _D = 4, 512, 2880
_N, _K, _H = 64, 8, 64
_G = _N // _K
_WINDOW = 128
_SM_SCALE = 1.0 / _H ** 0.5
_THETA = 150000.0
_INIT_CTX = 4096.0
_SCALING = 32.0
_NTK_ALPHA = 1.0
_NTK_BETA = 32.0
_NEG = -1e30

_T = _B * _L
_QC = _N * _H
_KC = _K * _H
_C_ALL = _QC + 2 * _KC
_ROPE_END = _QC + _KC


def _rope_tables(positions):
    d_half = _H // 2
    freq = _THETA ** (jnp.arange(0, _H, 2, dtype=jnp.float32) / _H)
    concentration = 0.1 * np.log(_SCALING) + 1.0
    low = d_half * np.log(_INIT_CTX / (_NTK_BETA * 2 * np.pi)) / np.log(_THETA)
    high = d_half * np.log(_INIT_CTX / (_NTK_ALPHA * 2 * np.pi)) / np.log(_THETA)
    interp = 1.0 / (_SCALING * freq)
    extrap = 1.0 / freq
    ramp = jnp.clip((jnp.arange(d_half, dtype=jnp.float32) - low) / (high - low), 0.0, 1.0)
    blend = 1.0 - ramp
    inv_freq = interp * (1.0 - blend) + extrap * blend
    ang = positions.astype(jnp.float32)[:, None] * inv_freq[None, :]
    return jnp.cos(ang) * concentration, jnp.sin(ang) * concentration


def _qkv_body(x_ref, w_ref, b_ref, cos_ref, sin_ref, o_ref):
    j = pl.program_id(0)
    h = jnp.dot(x_ref[...], w_ref[...], preferred_element_type=jnp.float32)
    h = h + b_ref[...]
    t, c = h.shape
    nh = c // _H
    h4 = h.reshape(t, nh, 2, _H // 2)
    cs = cos_ref[...][:, None, :]
    sn = sin_ref[...][:, None, :]
    x1 = h4[:, :, 0, :]
    x2 = h4[:, :, 1, :]
    r1 = x1 * cs - x2 * sn
    r2 = x2 * cs + x1 * sn
    roped = jnp.concatenate([r1[:, :, None, :], r2[:, :, None, :]], axis=2)
    roped = roped.reshape(t, c)
    col = j * c + jax.lax.broadcasted_iota(jnp.int32, (1, c), 1)
    o_ref[...] = jnp.where(col < _ROPE_END, roped, h).astype(jnp.bfloat16)


def _attn_body(q_ref, k_ref, v_ref, sinks_ref, o_ref):
    k_id = pl.program_id(1)
    kk = k_ref[:, 0, 0, :]
    vv = v_ref[:, 0, 0, :]
    kt = kk.T
    row = jax.lax.broadcasted_iota(jnp.int32, (_L, _L), 0)
    colm = jax.lax.broadcasted_iota(jnp.int32, (_L, _L), 1)
    allowed = (row >= colm) & (row - colm < _WINDOW)
    for g in range(_G):
        qg = q_ref[:, g, 0, :]
        s = jnp.dot(qg, kt, preferred_element_type=jnp.float32) * _SM_SCALE
        s = jnp.where(allowed, s, _NEG)
        sk = sinks_ref[k_id * _G + g]
        mrow = jnp.max(s, axis=-1)
        mf = jnp.maximum(mrow, sk)
        p = jnp.exp(s - mf[:, None])
        denom = jnp.exp(sk - mf) + jnp.sum(p, axis=-1)
        o = jnp.dot(p.astype(jnp.bfloat16), vv,
                    preferred_element_type=jnp.float32)
        o_ref[:, g, 0, :] = (o / denom[:, None]).astype(jnp.bfloat16)


def _proj_body(a_ref, w_ref, b_ref, o_ref):
    o_ref[...] = jnp.dot(a_ref[...], w_ref[...],
                         preferred_element_type=jnp.float32) + b_ref[...]


def kernel(x, wq, bq, wk, bk, wv, bv, wo, bo, sinks, positions):
    bf = jnp.bfloat16
    wqkv = jnp.concatenate(
        [wq.reshape(_D, _QC).astype(bf), wk.reshape(_D, _KC).astype(bf),
         wv.reshape(_D, _KC).astype(bf)], axis=1)
    bqkv = jnp.concatenate(
        [bq.reshape(_QC), bk.reshape(_KC), bv.reshape(_KC)])[None, :]
    cos, sin = _rope_tables(positions)

    ctile = 512
    qkv = pl.pallas_call(
        _qkv_body,
        out_shape=jax.ShapeDtypeStruct((_T, _C_ALL), bf),
        grid=(_C_ALL // ctile,),
        in_specs=[
            pl.BlockSpec(memory_space=pltpu.VMEM),
            pl.BlockSpec((_D, ctile), lambda j: (0, j)),
            pl.BlockSpec((1, ctile), lambda j: (0, j)),
            pl.BlockSpec(memory_space=pltpu.VMEM),
            pl.BlockSpec(memory_space=pltpu.VMEM),
        ],
        out_specs=pl.BlockSpec((_T, ctile), lambda j: (0, j)),
        compiler_params=pltpu.CompilerParams(
            dimension_semantics=("parallel",),
            vmem_limit_bytes=56 * 1024 * 1024,
        ),
        name="qkv_rope",
    )(x.astype(bf), wqkv, bqkv, cos, sin)

    qkv4 = qkv.reshape(_T, _C_ALL // _H, 1, _H)
    attn = pl.pallas_call(
        _attn_body,
        out_shape=jax.ShapeDtypeStruct((_T, _N, 1, _H), bf),
        grid=(_B, _K),
        in_specs=[
            pl.BlockSpec((_L, _G, 1, _H), lambda b, k: (b, k, 0, 0)),
            pl.BlockSpec((_L, 1, 1, _H),
                         lambda b, k: (b, _QC // _H + k, 0, 0)),
            pl.BlockSpec((_L, 1, 1, _H),
                         lambda b, k: (b, _ROPE_END // _H + k, 0, 0)),
            pl.BlockSpec(memory_space=pltpu.SMEM),
        ],
        out_specs=pl.BlockSpec((_L, _G, 1, _H), lambda b, k: (b, k, 0, 0)),
        compiler_params=pltpu.CompilerParams(
            dimension_semantics=("parallel", "parallel"),
        ),
        name="swa_attn",
    )(qkv4, qkv4, qkv4, sinks)
    attn = attn.reshape(_T, _QC)

    ttile = 128
    y = pl.pallas_call(
        _proj_body,
        out_shape=jax.ShapeDtypeStruct((_T, _D), jnp.float32),
        grid=(_T // ttile,),
        in_specs=[
            pl.BlockSpec((ttile, _QC), lambda i: (i, 0)),
            pl.BlockSpec(memory_space=pltpu.VMEM),
            pl.BlockSpec(memory_space=pltpu.VMEM),
        ],
        out_specs=pl.BlockSpec((ttile, _D), lambda i: (i, 0)),
        compiler_params=pltpu.CompilerParams(
            dimension_semantics=("parallel",),
            vmem_limit_bytes=56 * 1024 * 1024,
        ),
        name="out_proj",
    )(attn, wo.reshape(_QC, _D).astype(bf), bo[None, :])
    return y


# speedup vs baseline: 1.3196x; 1.3196x over previous
import jax
import jax.numpy as jnp
import numpy as np
from jax.experimental import pallas as pl
from jax.experimental.pallas import tpu as pltpu

_B, _L, _D = 4, 512, 2880
_N, _K, _H = 64, 8, 64
_G = _N // _K
_WINDOW = 128
_SM_SCALE = 1.0 / _H ** 0.5
_THETA = 150000.0
_INIT_CTX = 4096.0
_SCALING = 32.0
_NTK_ALPHA = 1.0
_NTK_BETA = 32.0
_NEG = -1e30

_T = _B * _L
_QC = _N * _H
_KC = _K * _H
_C_ALL = _QC + 2 * _KC
_ROPE_END = _QC + _KC


def _rope_tables(positions):
    d_half = _H // 2
    freq = _THETA ** (jnp.arange(0, _H, 2, dtype=jnp.float32) / _H)
    concentration = 0.1 * np.log(_SCALING) + 1.0
    low = d_half * np.log(_INIT_CTX / (_NTK_BETA * 2 * np.pi)) / np.log(_THETA)
    high = d_half * np.log(_INIT_CTX / (_NTK_ALPHA * 2 * np.pi)) / np.log(_THETA)
    interp = 1.0 / (_SCALING * freq)
    extrap = 1.0 / freq
    ramp = jnp.clip((jnp.arange(d_half, dtype=jnp.float32) - low) / (high - low), 0.0, 1.0)
    blend = 1.0 - ramp
    inv_freq = interp * (1.0 - blend) + extrap * blend
    ang = positions.astype(jnp.float32)[:, None] * inv_freq[None, :]
    return jnp.cos(ang) * concentration, jnp.sin(ang) * concentration


def _qkv_body(x_ref, w_ref, b_ref, cf_ref, sf_ref, p_ref, o_ref):
    j = pl.program_id(0)
    h = jnp.dot(x_ref[...], w_ref[...], preferred_element_type=jnp.float32)
    h = h + b_ref[...]
    t, c = h.shape
    partner = jnp.dot(h.astype(jnp.bfloat16), p_ref[...],
                      preferred_element_type=jnp.float32)
    roped = h * cf_ref[...] + partner * sf_ref[...]
    col = j * c + jax.lax.broadcasted_iota(jnp.int32, (1, c), 1)
    o_ref[...] = jnp.where(col < _ROPE_END, roped, h).astype(jnp.bfloat16)


def _attn_body(q_ref, k_ref, v_ref, sinks_ref, o_ref):
    k_id = pl.program_id(1)
    kk = k_ref[:, 0, 0, :]
    vv = v_ref[:, 0, 0, :]
    kt = kk.T
    row = jax.lax.broadcasted_iota(jnp.int32, (_L, _L), 0)
    colm = jax.lax.broadcasted_iota(jnp.int32, (_L, _L), 1)
    allowed = (row >= colm) & (row - colm < _WINDOW)
    for g in range(_G):
        qg = q_ref[:, g, 0, :]
        s = jnp.dot(qg, kt, preferred_element_type=jnp.float32) * _SM_SCALE
        s = jnp.where(allowed, s, _NEG)
        sk = sinks_ref[k_id * _G + g]
        mrow = jnp.max(s, axis=-1)
        mf = jnp.maximum(mrow, sk)
        p = jnp.exp(s - mf[:, None])
        denom = jnp.exp(sk - mf) + jnp.sum(p, axis=-1)
        o = jnp.dot(p.astype(jnp.bfloat16), vv,
                    preferred_element_type=jnp.float32)
        o_ref[:, g, 0, :] = (o / denom[:, None]).astype(jnp.bfloat16)


def _proj_body(a_ref, w_ref, b_ref, o_ref):
    o_ref[...] = jnp.dot(a_ref[...], w_ref[...],
                         preferred_element_type=jnp.float32) + b_ref[...]


def kernel(x, wq, bq, wk, bk, wv, bv, wo, bo, sinks, positions):
    bf = jnp.bfloat16
    wqkv = jnp.concatenate(
        [wq.reshape(_D, _QC).astype(bf), wk.reshape(_D, _KC).astype(bf),
         wv.reshape(_D, _KC).astype(bf)], axis=1)
    bqkv = jnp.concatenate(
        [bq.reshape(_QC), bk.reshape(_KC), bv.reshape(_KC)])[None, :]
    cos, sin = _rope_tables(positions)

    ctile = 512
    reps = ctile // _H
    cf = jnp.tile(jnp.concatenate([cos, cos], axis=1), (1, reps))
    sf = jnp.tile(jnp.concatenate([-sin, sin], axis=1), (1, reps))
    ia = jnp.arange(ctile)
    pmat = ((ia[:, None] // _H == ia[None, :] // _H)
            & (ia[:, None] % _H == (ia[None, :] + _H // 2) % _H)
            ).astype(bf)
    qkv = pl.pallas_call(
        _qkv_body,
        out_shape=jax.ShapeDtypeStruct((_T, _C_ALL), bf),
        grid=(_C_ALL // ctile,),
        in_specs=[
            pl.BlockSpec(memory_space=pltpu.VMEM),
            pl.BlockSpec((_D, ctile), lambda j: (0, j)),
            pl.BlockSpec((1, ctile), lambda j: (0, j)),
            pl.BlockSpec(memory_space=pltpu.VMEM),
            pl.BlockSpec(memory_space=pltpu.VMEM),
            pl.BlockSpec(memory_space=pltpu.VMEM),
        ],
        out_specs=pl.BlockSpec((_T, ctile), lambda j: (0, j)),
        compiler_params=pltpu.CompilerParams(
            dimension_semantics=("parallel",),
            vmem_limit_bytes=56 * 1024 * 1024,
        ),
        name="qkv_rope",
    )(x.astype(bf), wqkv, bqkv, cf, sf, pmat)

    qkv4 = qkv.reshape(_T, _C_ALL // _H, 1, _H)
    attn = pl.pallas_call(
        _attn_body,
        out_shape=jax.ShapeDtypeStruct((_T, _N, 1, _H), bf),
        grid=(_B, _K),
        in_specs=[
            pl.BlockSpec((_L, _G, 1, _H), lambda b, k: (b, k, 0, 0)),
            pl.BlockSpec((_L, 1, 1, _H),
                         lambda b, k: (b, _QC // _H + k, 0, 0)),
            pl.BlockSpec((_L, 1, 1, _H),
                         lambda b, k: (b, _ROPE_END // _H + k, 0, 0)),
            pl.BlockSpec(memory_space=pltpu.SMEM),
        ],
        out_specs=pl.BlockSpec((_L, _G, 1, _H), lambda b, k: (b, k, 0, 0)),
        compiler_params=pltpu.CompilerParams(
            dimension_semantics=("parallel", "parallel"),
        ),
        name="swa_attn",
    )(qkv4, qkv4, qkv4, sinks)
    attn = attn.reshape(_T, _QC)

    ttile = 128
    y = pl.pallas_call(
        _proj_body,
        out_shape=jax.ShapeDtypeStruct((_T, _D), jnp.float32),
        grid=(_T // ttile,),
        in_specs=[
            pl.BlockSpec((ttile, _QC), lambda i: (i, 0)),
            pl.BlockSpec(memory_space=pltpu.VMEM),
            pl.BlockSpec(memory_space=pltpu.VMEM),
        ],
        out_specs=pl.BlockSpec((ttile, _D), lambda i: (i, 0)),
        compiler_params=pltpu.CompilerParams(
            dimension_semantics=("parallel",),
            vmem_limit_bytes=56 * 1024 * 1024,
        ),
        name="out_proj",
    )(attn, wo.reshape(_QC, _D).astype(bf), bo[None, :])
    return y


# speedup vs baseline: 1.6455x; 1.2470x over previous
import jax
import jax.numpy as jnp
import numpy as np
from jax.experimental import pallas as pl
from jax.experimental.pallas import tpu as pltpu

_B, _L, _D = 4, 512, 2880
_N, _K, _H = 64, 8, 64
_G = _N // _K
_WINDOW = 128
_SM_SCALE = 1.0 / _H ** 0.5
_THETA = 150000.0
_INIT_CTX = 4096.0
_SCALING = 32.0
_NTK_ALPHA = 1.0
_NTK_BETA = 32.0
_NEG = -1e30

_T = _B * _L
_QC = _N * _H
_KC = _K * _H
_C_ALL = _QC + 2 * _KC
_ROPE_END = _QC + _KC


def _rope_tables(positions):
    d_half = _H // 2
    freq = _THETA ** (jnp.arange(0, _H, 2, dtype=jnp.float32) / _H)
    concentration = 0.1 * np.log(_SCALING) + 1.0
    low = d_half * np.log(_INIT_CTX / (_NTK_BETA * 2 * np.pi)) / np.log(_THETA)
    high = d_half * np.log(_INIT_CTX / (_NTK_ALPHA * 2 * np.pi)) / np.log(_THETA)
    interp = 1.0 / (_SCALING * freq)
    extrap = 1.0 / freq
    ramp = jnp.clip((jnp.arange(d_half, dtype=jnp.float32) - low) / (high - low), 0.0, 1.0)
    blend = 1.0 - ramp
    inv_freq = interp * (1.0 - blend) + extrap * blend
    ang = positions.astype(jnp.float32)[:, None] * inv_freq[None, :]
    return jnp.cos(ang) * concentration, jnp.sin(ang) * concentration


def _qkv_body(x_ref, w_ref, b_ref, cf_ref, sf_ref, p_ref, o_ref):
    j = pl.program_id(0)
    h = jnp.dot(x_ref[...], w_ref[...], preferred_element_type=jnp.float32)
    h = h + b_ref[...]
    t, c = h.shape
    partner = jnp.dot(h.astype(jnp.bfloat16), p_ref[...],
                      preferred_element_type=jnp.float32)
    roped = h * cf_ref[...] + partner * sf_ref[...]
    col = j * c + jax.lax.broadcasted_iota(jnp.int32, (1, c), 1)
    o_ref[...] = jnp.where(col < _ROPE_END, roped, h).astype(jnp.bfloat16)


def _attn_proj_body(q_ref, k_ref, v_ref, mb_ref, sinks_ref, wo_ref, bo_ref,
                    o_ref, asc_ref):
    k_id = pl.program_id(1)
    kk = k_ref[:, 0, 0, :]
    vv = v_ref[:, 0, 0, :]
    kt = kk.T
    ones = jnp.ones((_L, _H), jnp.bfloat16)
    vva = jnp.concatenate([vv, ones], axis=1)
    mb = mb_ref[...]
    for g in range(_G):
        qg = q_ref[:, g, 0, :]
        s = jnp.dot(qg, kt, preferred_element_type=jnp.float32) + mb
        sk = sinks_ref[k_id * _G + g]
        mf = jnp.maximum(jnp.max(s, axis=-1), sk)
        p = jnp.exp(s - mf[:, None])
        pv = jnp.dot(p.astype(jnp.bfloat16), vva,
                     preferred_element_type=jnp.float32)
        denom = jnp.exp(sk - mf) + pv[:, _H]
        og = pv[:, :_H] / denom[:, None]
        asc_ref[:, g * _H:(g + 1) * _H] = og.astype(jnp.bfloat16)
    wos = wo_ref[pl.ds(k_id * _G * _H, _G * _H), :]
    partial = jnp.dot(asc_ref[...], wos,
                      preferred_element_type=jnp.float32)

    @pl.when(k_id == 0)
    def _():
        o_ref[...] = partial + bo_ref[...]

    @pl.when(k_id != 0)
    def _():
        o_ref[...] += partial


def kernel(x, wq, bq, wk, bk, wv, bv, wo, bo, sinks, positions):
    bf = jnp.bfloat16
    wqkv = jnp.concatenate(
        [(wq.reshape(_D, _QC) * _SM_SCALE).astype(bf),
         wk.reshape(_D, _KC).astype(bf),
         wv.reshape(_D, _KC).astype(bf)], axis=1)
    bqkv = jnp.concatenate(
        [bq.reshape(_QC) * _SM_SCALE, bk.reshape(_KC), bv.reshape(_KC)])[None, :]
    cos, sin = _rope_tables(positions)

    ctile = 512
    reps = ctile // _H
    cf = jnp.tile(jnp.concatenate([cos, cos], axis=1), (1, reps))
    sf = jnp.tile(jnp.concatenate([-sin, sin], axis=1), (1, reps))
    ia = jnp.arange(ctile)
    pmat = ((ia[:, None] // _H == ia[None, :] // _H)
            & (ia[:, None] % _H == (ia[None, :] + _H // 2) % _H)
            ).astype(bf)
    qkv = pl.pallas_call(
        _qkv_body,
        out_shape=jax.ShapeDtypeStruct((_T, _C_ALL), bf),
        grid=(_C_ALL // ctile,),
        in_specs=[
            pl.BlockSpec(memory_space=pltpu.VMEM),
            pl.BlockSpec((_D, ctile), lambda j: (0, j)),
            pl.BlockSpec((1, ctile), lambda j: (0, j)),
            pl.BlockSpec(memory_space=pltpu.VMEM),
            pl.BlockSpec(memory_space=pltpu.VMEM),
            pl.BlockSpec(memory_space=pltpu.VMEM),
        ],
        out_specs=pl.BlockSpec((_T, ctile), lambda j: (0, j)),
        compiler_params=pltpu.CompilerParams(
            dimension_semantics=("parallel",),
            vmem_limit_bytes=56 * 1024 * 1024,
        ),
        name="qkv_rope",
    )(x.astype(bf), wqkv, bqkv, cf, sf, pmat)

    qkv4 = qkv.reshape(_T, _C_ALL // _H, 1, _H)
    pos = jnp.arange(_L)
    delta = pos[:, None] - pos[None, :]
    mbias = jnp.where((delta >= 0) & (delta < _WINDOW), 0.0, _NEG
                      ).astype(jnp.float32)
    y = pl.pallas_call(
        _attn_proj_body,
        out_shape=jax.ShapeDtypeStruct((_T, _D), jnp.float32),
        grid=(_B, _K),
        in_specs=[
            pl.BlockSpec((_L, _G, 1, _H), lambda b, k: (b, k, 0, 0)),
            pl.BlockSpec((_L, 1, 1, _H),
                         lambda b, k: (b, _QC // _H + k, 0, 0)),
            pl.BlockSpec((_L, 1, 1, _H),
                         lambda b, k: (b, _ROPE_END // _H + k, 0, 0)),
            pl.BlockSpec(memory_space=pltpu.VMEM),
            pl.BlockSpec(memory_space=pltpu.SMEM),
            pl.BlockSpec(memory_space=pltpu.VMEM),
            pl.BlockSpec(memory_space=pltpu.VMEM),
        ],
        out_specs=pl.BlockSpec((_L, _D), lambda b, k: (b, 0)),
        scratch_shapes=[pltpu.VMEM((_L, _G * _H), bf)],
        compiler_params=pltpu.CompilerParams(
            dimension_semantics=("parallel", "arbitrary"),
            vmem_limit_bytes=56 * 1024 * 1024,
        ),
        name="swa_attn_proj",
    )(qkv4, qkv4, qkv4, mbias, sinks, wo.reshape(_QC, _D).astype(bf),
      bo[None, :])
    return y


# speedup vs baseline: 1.7062x; 1.0369x over previous
import jax
import jax.numpy as jnp
import numpy as np
from jax.experimental import pallas as pl
from jax.experimental.pallas import tpu as pltpu

_B, _L, _D = 4, 512, 2880
_N, _K, _H = 64, 8, 64
_G = _N // _K
_WINDOW = 128
_SM_SCALE = 1.0 / _H ** 0.5
_THETA = 150000.0
_INIT_CTX = 4096.0
_SCALING = 32.0
_NTK_ALPHA = 1.0
_NTK_BETA = 32.0
_NEG = -1e30

_T = _B * _L
_QC = _N * _H
_KC = _K * _H
_C_ALL = _QC + 2 * _KC
_ROPE_END = _QC + _KC


def _rope_tables(positions):
    d_half = _H // 2
    freq = _THETA ** (jnp.arange(0, _H, 2, dtype=jnp.float32) / _H)
    concentration = 0.1 * np.log(_SCALING) + 1.0
    low = d_half * np.log(_INIT_CTX / (_NTK_BETA * 2 * np.pi)) / np.log(_THETA)
    high = d_half * np.log(_INIT_CTX / (_NTK_ALPHA * 2 * np.pi)) / np.log(_THETA)
    interp = 1.0 / (_SCALING * freq)
    extrap = 1.0 / freq
    ramp = jnp.clip((jnp.arange(d_half, dtype=jnp.float32) - low) / (high - low), 0.0, 1.0)
    blend = 1.0 - ramp
    inv_freq = interp * (1.0 - blend) + extrap * blend
    ang = positions.astype(jnp.float32)[:, None] * inv_freq[None, :]
    return jnp.cos(ang) * concentration, jnp.sin(ang) * concentration


def _qkv_body(x_ref, w_ref, b_ref, cf_ref, sf_ref, p_ref, o_ref):
    j = pl.program_id(0)
    h = jnp.dot(x_ref[...], w_ref[...], preferred_element_type=jnp.float32)
    h = h + b_ref[...]
    t, c = h.shape
    partner = jnp.dot(h.astype(jnp.bfloat16), p_ref[...],
                      preferred_element_type=jnp.float32)
    roped = h * cf_ref[...] + partner * sf_ref[...]
    col = j * c + jax.lax.broadcasted_iota(jnp.int32, (1, c), 1)
    o_ref[...] = jnp.where(col < _ROPE_END, roped, h).astype(jnp.bfloat16)


def _attn_proj_body(q_ref, k_ref, v_ref, mb_ref, sinks_ref, wo_ref, bo_ref,
                    o_ref, asc_ref):
    k_id = pl.program_id(1)
    kk = k_ref[:, 0, 0, :]
    vv = v_ref[:, 0, 0, :]
    vvat = jnp.concatenate(
        [vv.T, jnp.ones((_H, _L), jnp.bfloat16)], axis=0)
    qt = q_ref[:, :, 0, :].reshape(_L, _G * _H).T
    mbt = mb_ref[...]
    for g in range(_G):
        qgt = qt[g * _H:(g + 1) * _H, :]
        st = jnp.dot(kk, qgt, preferred_element_type=jnp.float32) + mbt
        sk = sinks_ref[k_id * _G + g]
        mf = jnp.maximum(jnp.max(st, axis=0), sk)
        pt = jnp.exp(st - mf[None, :])
        pvt = jnp.dot(vvat, pt.astype(jnp.bfloat16),
                      preferred_element_type=jnp.float32)
        denom = jnp.exp(sk - mf) + pvt[_H, :]
        ogt = pvt[:_H, :] / denom[None, :]
        asc_ref[g * _H:(g + 1) * _H, :] = ogt.astype(jnp.bfloat16)
    wos = wo_ref[pl.ds(k_id * _G * _H, _G * _H), :]
    partial = jax.lax.dot_general(
        asc_ref[...], wos, (((0,), (0,)), ((), ())),
        preferred_element_type=jnp.float32)

    @pl.when(k_id == 0)
    def _():
        o_ref[...] = partial + bo_ref[...]

    @pl.when(k_id != 0)
    def _():
        o_ref[...] += partial


def kernel(x, wq, bq, wk, bk, wv, bv, wo, bo, sinks, positions):
    bf = jnp.bfloat16
    wqkv = jnp.concatenate(
        [(wq.reshape(_D, _QC) * _SM_SCALE).astype(bf),
         wk.reshape(_D, _KC).astype(bf),
         wv.reshape(_D, _KC).astype(bf)], axis=1)
    bqkv = jnp.concatenate(
        [bq.reshape(_QC) * _SM_SCALE, bk.reshape(_KC), bv.reshape(_KC)])[None, :]
    cos, sin = _rope_tables(positions)

    ctile = 512
    reps = ctile // _H
    cf = jnp.tile(jnp.concatenate([cos, cos], axis=1), (1, reps))
    sf = jnp.tile(jnp.concatenate([-sin, sin], axis=1), (1, reps))
    ia = jnp.arange(ctile)
    pmat = ((ia[:, None] // _H == ia[None, :] // _H)
            & (ia[:, None] % _H == (ia[None, :] + _H // 2) % _H)
            ).astype(bf)
    qkv = pl.pallas_call(
        _qkv_body,
        out_shape=jax.ShapeDtypeStruct((_T, _C_ALL), bf),
        grid=(_C_ALL // ctile,),
        in_specs=[
            pl.BlockSpec(memory_space=pltpu.VMEM),
            pl.BlockSpec((_D, ctile), lambda j: (0, j)),
            pl.BlockSpec((1, ctile), lambda j: (0, j)),
            pl.BlockSpec(memory_space=pltpu.VMEM),
            pl.BlockSpec(memory_space=pltpu.VMEM),
            pl.BlockSpec(memory_space=pltpu.VMEM),
        ],
        out_specs=pl.BlockSpec((_T, ctile), lambda j: (0, j)),
        compiler_params=pltpu.CompilerParams(
            dimension_semantics=("parallel",),
            vmem_limit_bytes=56 * 1024 * 1024,
        ),
        name="qkv_rope",
    )(x.astype(bf), wqkv, bqkv, cf, sf, pmat)

    qkv4 = qkv.reshape(_T, _C_ALL // _H, 1, _H)
    pos = jnp.arange(_L)
    delta = pos[None, :] - pos[:, None]
    mbias = jnp.where((delta >= 0) & (delta < _WINDOW), 0.0, _NEG
                      ).astype(jnp.float32)
    y = pl.pallas_call(
        _attn_proj_body,
        out_shape=jax.ShapeDtypeStruct((_T, _D), jnp.float32),
        grid=(_B, _K),
        in_specs=[
            pl.BlockSpec((_L, _G, 1, _H), lambda b, k: (b, k, 0, 0)),
            pl.BlockSpec((_L, 1, 1, _H),
                         lambda b, k: (b, _QC // _H + k, 0, 0)),
            pl.BlockSpec((_L, 1, 1, _H),
                         lambda b, k: (b, _ROPE_END // _H + k, 0, 0)),
            pl.BlockSpec(memory_space=pltpu.VMEM),
            pl.BlockSpec(memory_space=pltpu.SMEM),
            pl.BlockSpec(memory_space=pltpu.VMEM),
            pl.BlockSpec(memory_space=pltpu.VMEM),
        ],
        out_specs=pl.BlockSpec((_L, _D), lambda b, k: (b, 0)),
        scratch_shapes=[pltpu.VMEM((_G * _H, _L), bf)],
        compiler_params=pltpu.CompilerParams(
            dimension_semantics=("parallel", "arbitrary"),
            vmem_limit_bytes=56 * 1024 * 1024,
        ),
        name="swa_attn_proj",
    )(qkv4, qkv4, qkv4, mbias, sinks, wo.reshape(_QC, _D).astype(bf),
      bo[None, :])
    return y


# speedup vs baseline: 1.8528x; 1.0859x over previous
import jax
import jax.numpy as jnp
import numpy as np
from jax.experimental import pallas as pl
from jax.experimental.pallas import tpu as pltpu

_B, _L, _D = 4, 512, 2880
_N, _K, _H = 64, 8, 64
_G = _N // _K
_WINDOW = 128
_SM_SCALE = 1.0 / _H ** 0.5
_THETA = 150000.0
_INIT_CTX = 4096.0
_SCALING = 32.0
_NTK_ALPHA = 1.0
_NTK_BETA = 32.0
_NEG = -1e30

_T = _B * _L
_QC = _N * _H
_KC = _K * _H
_C_ALL = _QC + 2 * _KC
_ROPE_END = _QC + _KC


def _rope_tables(positions):
    d_half = _H // 2
    freq = _THETA ** (jnp.arange(0, _H, 2, dtype=jnp.float32) / _H)
    concentration = 0.1 * np.log(_SCALING) + 1.0
    low = d_half * np.log(_INIT_CTX / (_NTK_BETA * 2 * np.pi)) / np.log(_THETA)
    high = d_half * np.log(_INIT_CTX / (_NTK_ALPHA * 2 * np.pi)) / np.log(_THETA)
    interp = 1.0 / (_SCALING * freq)
    extrap = 1.0 / freq
    ramp = jnp.clip((jnp.arange(d_half, dtype=jnp.float32) - low) / (high - low), 0.0, 1.0)
    blend = 1.0 - ramp
    inv_freq = interp * (1.0 - blend) + extrap * blend
    ang = positions.astype(jnp.float32)[:, None] * inv_freq[None, :]
    return jnp.cos(ang) * concentration, jnp.sin(ang) * concentration


def _qkv_body(x_ref, w_ref, b_ref, cf_ref, sf_ref, p_ref, o_ref):
    j = pl.program_id(0)
    h = jnp.dot(x_ref[...], w_ref[...], preferred_element_type=jnp.float32)
    h = h + b_ref[...]
    t, c = h.shape
    partner = jnp.dot(h.astype(jnp.bfloat16), p_ref[...],
                      preferred_element_type=jnp.float32)
    roped = h * cf_ref[...] + partner * sf_ref[...]
    col = j * c + jax.lax.broadcasted_iota(jnp.int32, (1, c), 1)
    o_ref[...] = jnp.where(col < _ROPE_END, roped, h).astype(jnp.bfloat16)


def _attn_proj_body(q_ref, k_ref, v_ref, mb_ref, sink_ref, wo_ref, bo_ref,
                    o_ref, asc_ref):
    k_id = pl.program_id(1)
    kk = k_ref[:, 0, 0, :]
    vv = v_ref[:, 0, 0, :]
    vvat = jnp.concatenate(
        [vv.T, jnp.ones((_H, _L), jnp.bfloat16)], axis=0)
    qt = q_ref[:, :, 0, :].reshape(_L, _G * _H).T
    sks = sink_ref[0, 0, :]
    C = 128
    S = 2 * C
    nc = _L // C
    for c in range(nc):
        ms = 0 if c == 0 else (c - 1) * C
        kc = kk[ms:ms + S, :]
        rhs = jnp.concatenate(
            [qt[g * _H:(g + 1) * _H, c * C:(c + 1) * C] for g in range(_G)],
            axis=1)
        mb = mb_ref[0 if c == 0 else 1]
        st = jnp.dot(kc, rhs, preferred_element_type=jnp.float32) + mb
        mf = jnp.maximum(jnp.max(st, axis=0), sks)
        pt = jnp.exp(st - mf[None, :])
        ot = jnp.dot(vvat[:, ms:ms + S], pt.astype(jnp.bfloat16),
                     preferred_element_type=jnp.float32)
        denom = jnp.exp(sks - mf) + ot[_H, :]
        ogt = (ot[:_H, :] / denom[None, :]).astype(jnp.bfloat16)
        for g in range(_G):
            asc_ref[g * _H:(g + 1) * _H, c * C:(c + 1) * C] = \
                ogt[:, g * C:(g + 1) * C]
    wos = wo_ref[pl.ds(k_id * _G * _H, _G * _H), :]
    partial = jax.lax.dot_general(
        asc_ref[...], wos, (((0,), (0,)), ((), ())),
        preferred_element_type=jnp.float32)

    @pl.when(k_id == 0)
    def _():
        o_ref[...] = partial + bo_ref[...]

    @pl.when(k_id != 0)
    def _():
        o_ref[...] += partial


def kernel(x, wq, bq, wk, bk, wv, bv, wo, bo, sinks, positions):
    bf = jnp.bfloat16
    wqkv = jnp.concatenate(
        [(wq.reshape(_D, _QC) * _SM_SCALE).astype(bf),
         wk.reshape(_D, _KC).astype(bf),
         wv.reshape(_D, _KC).astype(bf)], axis=1)
    bqkv = jnp.concatenate(
        [bq.reshape(_QC) * _SM_SCALE, bk.reshape(_KC), bv.reshape(_KC)])[None, :]
    cos, sin = _rope_tables(positions)

    ctile = 512
    reps = ctile // _H
    cf = jnp.tile(jnp.concatenate([cos, cos], axis=1), (1, reps))
    sf = jnp.tile(jnp.concatenate([-sin, sin], axis=1), (1, reps))
    ia = jnp.arange(ctile)
    pmat = ((ia[:, None] // _H == ia[None, :] // _H)
            & (ia[:, None] % _H == (ia[None, :] + _H // 2) % _H)
            ).astype(bf)
    qkv = pl.pallas_call(
        _qkv_body,
        out_shape=jax.ShapeDtypeStruct((_T, _C_ALL), bf),
        grid=(_C_ALL // ctile,),
        in_specs=[
            pl.BlockSpec(memory_space=pltpu.VMEM),
            pl.BlockSpec((_D, ctile), lambda j: (0, j)),
            pl.BlockSpec((1, ctile), lambda j: (0, j)),
            pl.BlockSpec(memory_space=pltpu.VMEM),
            pl.BlockSpec(memory_space=pltpu.VMEM),
            pl.BlockSpec(memory_space=pltpu.VMEM),
        ],
        out_specs=pl.BlockSpec((_T, ctile), lambda j: (0, j)),
        compiler_params=pltpu.CompilerParams(
            dimension_semantics=("parallel",),
            vmem_limit_bytes=56 * 1024 * 1024,
        ),
        name="qkv_rope",
    )(x.astype(bf), wqkv, bqkv, cf, sf, pmat)

    qkv4 = qkv.reshape(_T, _C_ALL // _H, 1, _H)
    cchunk = 128
    span = 2 * cchunk
    mr = jnp.arange(span)[:, None]
    lc = jnp.arange(cchunk)[None, :]
    d0 = lc - mr
    d1 = cchunk + lc - mr
    m0 = jnp.where((d0 >= 0) & (d0 < _WINDOW), 0.0, _NEG)
    m1 = jnp.where((d1 >= 0) & (d1 < _WINDOW), 0.0, _NEG)
    mbias = jnp.stack([jnp.tile(m0, (1, _G)), jnp.tile(m1, (1, _G))]
                      ).astype(jnp.float32)
    sinkmat = jnp.repeat(sinks.reshape(_K, _G), cchunk, axis=1
                         ).reshape(_K, 1, _G * cchunk)
    y = pl.pallas_call(
        _attn_proj_body,
        out_shape=jax.ShapeDtypeStruct((_T, _D), jnp.float32),
        grid=(_B, _K),
        in_specs=[
            pl.BlockSpec((_L, _G, 1, _H), lambda b, k: (b, k, 0, 0)),
            pl.BlockSpec((_L, 1, 1, _H),
                         lambda b, k: (b, _QC // _H + k, 0, 0)),
            pl.BlockSpec((_L, 1, 1, _H),
                         lambda b, k: (b, _ROPE_END // _H + k, 0, 0)),
            pl.BlockSpec(memory_space=pltpu.VMEM),
            pl.BlockSpec((1, 1, _G * cchunk), lambda b, k: (k, 0, 0)),
            pl.BlockSpec(memory_space=pltpu.VMEM),
            pl.BlockSpec(memory_space=pltpu.VMEM),
        ],
        out_specs=pl.BlockSpec((_L, _D), lambda b, k: (b, 0)),
        scratch_shapes=[pltpu.VMEM((_G * _H, _L), bf)],
        compiler_params=pltpu.CompilerParams(
            dimension_semantics=("parallel", "arbitrary"),
            vmem_limit_bytes=56 * 1024 * 1024,
        ),
        name="swa_attn_proj",
    )(qkv4, qkv4, qkv4, mbias, sinkmat, wo.reshape(_QC, _D).astype(bf),
      bo[None, :])
    return y
